# Initial kernel scaffold; baseline (speedup 1.0000x reference)
import jax
import jax.numpy as jnp
from jax.experimental import pallas as pl


def kernel(x, router_w, w1):
    raise NotImplementedError("write your pallas kernel here")


# speedup vs baseline: 1.3662x; 1.3662x over previous
import functools

import jax
import jax.numpy as jnp
from jax import lax
from jax.experimental import pallas as pl
from jax.experimental.pallas import tpu as pltpu
import jax.experimental.pallas.tpu_sc as plsc

E = 64
TOPK = 8
D = 768
F = 384
T = 2048
M = T * TOPK
BM = 128
NB = M // BM
TBLK = 256

NW = 32
JW = M // NW
TW = T // NW
L = 16


def _router_body(x_ref, rwt_ref, logits_ref, exp_ref, wts_ref):
    xb = x_ref[...]
    lg = jnp.dot(xb, rwt_ref[...], preferred_element_type=jnp.float32)
    logits_ref[...] = lg
    m = jnp.max(lg, axis=1, keepdims=True)
    p = jnp.exp(lg - m)
    r = p / jnp.sum(p, axis=1, keepdims=True)
    lane = lax.broadcasted_iota(jnp.int32, (TBLK, E), 1)
    vals, idxs = [], []
    for _ in range(TOPK):
        mk = jnp.max(r, axis=1, keepdims=True)
        ik = jnp.min(jnp.where(r == mk, lane, E), axis=1, keepdims=True)
        vals.append(mk)
        idxs.append(ik)
        r = jnp.where(lane == ik, -1.0, r)
    v = jnp.concatenate(vals, axis=1)
    wts_ref[...] = v / jnp.sum(v, axis=1, keepdims=True)
    exp_ref[...] = jnp.concatenate(idxs, axis=1)


def _router_call(x_flat, rwt, interpret=False):
    return pl.pallas_call(
        _router_body,
        grid=(T // TBLK,),
        in_specs=[
            pl.BlockSpec((TBLK, D), lambda i: (i, 0)),
            pl.BlockSpec((D, E), lambda i: (0, 0)),
        ],
        out_specs=[
            pl.BlockSpec((TBLK, E), lambda i: (i, 0)),
            pl.BlockSpec((TBLK, TOPK), lambda i: (i, 0)),
            pl.BlockSpec((TBLK, TOPK), lambda i: (i, 0)),
        ],
        out_shape=[
            jax.ShapeDtypeStruct((T, E), jnp.float32),
            jax.ShapeDtypeStruct((T, TOPK), jnp.int32),
            jax.ShapeDtypeStruct((T, TOPK), jnp.float32),
        ],
        interpret=interpret,
    )(x_flat, rwt)


def _sort_body(exp_ref, pos_ref, bexp_ref, rpre_ref):
    e_iota = lax.broadcasted_iota(jnp.int32, (E, BM), 0)
    r128 = lax.broadcasted_iota(jnp.int32, (BM, BM), 0)
    c128 = lax.broadcasted_iota(jnp.int32, (BM, BM), 1)
    tri = (r128 <= c128).astype(jnp.float32)
    r64 = lax.broadcasted_iota(jnp.int32, (E, E), 0)
    c64 = lax.broadcasted_iota(jnp.int32, (E, E), 1)
    tri_ex = (c64 < r64).astype(jnp.float32)

    def onehot(i):
        erow = exp_ref[pl.ds(i, 1), :]
        return (jnp.broadcast_to(erow, (E, BM)) == e_iota).astype(jnp.float32)

    def body1(i, carry):
        o = onehot(i)
        cum = jnp.dot(o, tri, preferred_element_type=jnp.float32)
        rank_incl = jnp.sum(cum * o, axis=0, keepdims=True)
        carry_sel = jnp.sum(carry * o, axis=0, keepdims=True)
        rpre_ref[pl.ds(i, 1), :] = carry_sel + rank_incl - 1.0
        return carry + jnp.sum(o, axis=1, keepdims=True)

    counts = lax.fori_loop(0, NB, body1, jnp.zeros((E, 1), jnp.float32))
    c_hi = jnp.floor(counts * (1.0 / 256.0))
    c_lo = counts - c_hi * 256.0
    hi_mat = jnp.broadcast_to(c_hi, (E, BM))
    lo_mat = jnp.broadcast_to(c_lo, (E, BM))
    offs_mat = (jnp.dot(tri_ex, hi_mat, preferred_element_type=jnp.float32) * 256.0
                + jnp.dot(tri_ex, lo_mat, preferred_element_type=jnp.float32))

    def body2(i, _):
        o = onehot(i)
        offs_sel = jnp.sum(offs_mat[:, :1] * o, axis=0, keepdims=True)
        pos = rpre_ref[pl.ds(i, 1), :] + offs_sel
        pos_ref[pl.ds(i, 1), :] = pos.astype(jnp.int32)
        return 0

    lax.fori_loop(0, NB, body2, 0)

    q = (128 * lax.broadcasted_iota(jnp.int32, (E, NB), 1)).astype(jnp.float32)
    cmp = (offs_mat[:, :NB] <= q).astype(jnp.float32)
    bexp_ref[...] = (jnp.sum(cmp, axis=0, keepdims=True) - 1.0).astype(jnp.int32)


def _sort_call(experts2d, interpret=False):
    return pl.pallas_call(
        _sort_body,
        out_shape=[
            jax.ShapeDtypeStruct((NB, BM), jnp.int32),
            jax.ShapeDtypeStruct((1, NB), jnp.int32),
        ],
        scratch_shapes=[pltpu.VMEM((NB, BM), jnp.float32)],
        interpret=interpret,
    )(experts2d)


def _sc_scatter_body(x_hbm, pos8_hbm, w8_hbm, xg_hbm, ws_hbm,
                     rowbuf, idxk, wk, sem):
    w = lax.axis_index("s") * 2 + lax.axis_index("c")
    tbase = w * TW
    pltpu.sync_copy(x_hbm.at[pl.ds(tbase, TW), :], rowbuf)
    for k in range(TOPK):
        pltpu.sync_copy(pos8_hbm.at[k, pl.ds(tbase, TW)], idxk.at[k])
        pltpu.sync_copy(w8_hbm.at[k, pl.ds(tbase, TW)], wk.at[k])
    copies = []
    for k in range(TOPK):
        copies.append(pltpu.async_copy(rowbuf, xg_hbm.at[idxk.at[k]], sem))
    for c in copies:
        c.wait()
    copies = []
    for k in range(TOPK):
        copies.append(pltpu.async_copy(wk.at[k], ws_hbm.at[idxk.at[k]], sem))
    for c in copies:
        c.wait()


def _sc_call(x_flat, pos8, w8):
    mesh = plsc.VectorSubcoreMesh(core_axis_name="c", subcore_axis_name="s")
    f = functools.partial(
        pl.kernel,
        out_type=(
            jax.ShapeDtypeStruct((M, D), jnp.float32),
            jax.ShapeDtypeStruct((M,), jnp.float32),
        ),
        mesh=mesh,
        scratch_types=[
            pltpu.VMEM((TW, D), jnp.float32),
            pltpu.VMEM((TOPK, TW), jnp.int32),
            pltpu.VMEM((TOPK, TW), jnp.float32),
            pltpu.SemaphoreType.DMA,
        ],
    )(_sc_scatter_body)
    return f(x_flat, pos8, w8)


def _mm_body(bexp_ref, xg_ref, w1_ref, ws_ref, out_ref):
    acc = jnp.dot(xg_ref[...], w1_ref[...], preferred_element_type=jnp.float32)
    g = 0.5 * acc * (1.0 + lax.erf(acc * 0.7071067811865476))
    out_ref[...] = g * ws_ref[...]


def _mm_call(bexp, xg, w1, ws2d, interpret=False):
    grid_spec = pltpu.PrefetchScalarGridSpec(
        num_scalar_prefetch=1,
        grid=(NB,),
        in_specs=[
            pl.BlockSpec((BM, D), lambda d, be: (d, 0)),
            pl.BlockSpec((D, F), lambda d, be: (0, be[d])),
            pl.BlockSpec((BM, 1), lambda d, be: (d, 0)),
        ],
        out_specs=pl.BlockSpec((BM, F), lambda d, be: (d, 0)),
    )
    return pl.pallas_call(
        _mm_body,
        grid_spec=grid_spec,
        out_shape=jax.ShapeDtypeStruct((M, F), jnp.float32),
        interpret=interpret,
    )(bexp, xg, w1, ws2d)


def kernel(x, router_w, w1):
    B, S, Dm = x.shape
    x_flat = x.reshape(B * S, Dm)
    logits, experts, wts = _router_call(x_flat, router_w.T)
    pos2d, bexp = _sort_call(experts.reshape(NB, BM))
    xg, ws = _sc_call(x_flat, pos2d.reshape(T, TOPK).T, wts.T)
    out = _mm_call(bexp.reshape(NB), xg, w1, ws.reshape(M, 1))
    return out, logits
